# baseline (speedup 1.0000x reference)
import functools
import jax
import jax.numpy as jnp
from jax import lax
from jax.experimental import pallas as pl
from jax.experimental.pallas import tpu as pltpu
from jax.experimental.pallas import tpu_sc as plsc

N = 10000
E = 320000
IN_DIM = 128
HID = 64
OUT_DIM = 128
HEADS = 8
NUM_PAR = 32

NC = 2
NS = 16
CE = 128
NCH = 159
EP = NS * NCH * CE
NQ = 10240
NQT = NQ // NS
NB = 1024
GRID = NQ // NB

f32 = jnp.float32
i32 = jnp.int32


def _k1_body(h_ref, w1_ref, a1_ref, z_ref, e1_ref):
    z = jnp.dot(h_ref[...], w1_ref[...], preferred_element_type=f32)
    e1_ref[...] = lax.dot_general(a1_ref[...], z, (((0,), (1,)), ((), ())))
    for hd in range(HEADS):
        z_ref[hd] = z[:, hd * HID:(hd + 1) * HID]


def _k1(h, W1f, A1):
    return pl.pallas_call(
        _k1_body,
        grid=(GRID,),
        in_specs=[
            pl.BlockSpec((NB, IN_DIM), lambda i: (i, 0)),
            pl.BlockSpec((IN_DIM, HEADS * HID), lambda i: (0, 0)),
            pl.BlockSpec((HEADS * HID, 2 * HEADS), lambda i: (0, 0)),
        ],
        out_specs=[
            pl.BlockSpec((HEADS, NB, HID), lambda i: (0, i, 0)),
            pl.BlockSpec((2 * HEADS, NB), lambda i: (0, i)),
        ],
        out_shape=[
            jax.ShapeDtypeStruct((HEADS, NQ, HID), f32),
            jax.ShapeDtypeStruct((2 * HEADS, NQ), f32),
        ],
    )(h, W1f, A1)


def _edge_sweep(hN, srcv, dstv, esv, edv, z_r, acc_sh, den_sh, bufs):
    def front(g, b):
        exv, idxv, zb, sg, sa, sd = bufs[b]
        for j in range(CE // 16):
            sv = srcv[g, pl.ds(j * 16, 16)]
            dv = dstv[g, pl.ds(j * 16, 16)]
            e = plsc.load_gather(esv, [sv]) + plsc.load_gather(edv, [dv])
            e = jnp.where(e >= 0, e, 0.2 * e)
            exv[pl.ds(j * 16, 16)] = jnp.exp(e)
            idxv[pl.ds(j * 16, 16)] = sv + hN
        pltpu.async_copy(z_r.at[idxv], zb, sg)

    def wait_scat(b, g):
        exv, idxv, zb, sg, sa, sd = bufs[b]
        pltpu.make_async_copy(zb, acc_sh.at[dstv.at[g]], sa).wait()
        pltpu.make_async_copy(exv, den_sh.at[dstv.at[g]], sd).wait()

    def back(g, b):
        exv, idxv, zb, sg, sa, sd = bufs[b]
        pltpu.make_async_copy(z_r.at[idxv], zb, sg).wait()

        def scale(jj, cc):
            ex16 = exv[pl.ds(jj * 16, 16)]
            for rr in range(16):
                sc = ex16[rr]
                r = jj * 16 + rr
                for k in range(HID // 16):
                    zb[r, pl.ds(k * 16, 16)] = zb[r, pl.ds(k * 16, 16)] * sc
            return cc
        lax.fori_loop(0, CE // 16, scale, 0)
        pltpu.async_copy(zb, acc_sh.at[dstv.at[g]], sa, add=True)
        pltpu.async_copy(exv, den_sh.at[dstv.at[g]], sd, add=True)

    front(0, 0)
    front(1, 1)
    back(0, 0)
    front(2, 2)
    back(1, 1)

    def main(p, cc):
        for off in range(3):
            g = 3 * p + 3 + off
            wait_scat(off, g)
            front(g, off)
            back(g - 1, (off + 2) % 3)
        return cc
    lax.fori_loop(0, (NCH - 3) // 3, main, 0)
    back(NCH - 1, (NCH - 1) % 3)
    for b in range(3):
        wait_scat(b, 0)


def _k2_body(src_r, dst_r, e1_r, z_r, zacc_r, zden_r, acc_r, den_r,
             srcv, dstv, esv, edv,
             exv0, idxv0, zb0, exv1, idxv1, zb1, exv2, idxv2, zb2,
             acc_sh, den_sh,
             sg0, sa0, sd0, sg1, sa1, sd1, sg2, sa2, sd2):
    c = lax.axis_index("c")
    s = lax.axis_index("s")
    pltpu.sync_copy(src_r.at[s], srcv)
    pltpu.sync_copy(dst_r.at[s], dstv)
    r0 = s * NQT
    bufs = [(exv0, idxv0, zb0, sg0, sa0, sd0),
            (exv1, idxv1, zb1, sg1, sa1, sd1),
            (exv2, idxv2, zb2, sg2, sa2, sd2)]

    def head(hi, cc):
        h = c * (HEADS // NC) + hi
        pltpu.sync_copy(e1_r.at[h], esv)
        pltpu.sync_copy(e1_r.at[HEADS + h], edv)
        pltpu.sync_copy(zacc_r.at[pl.ds(r0, NQT)], acc_sh.at[pl.ds(r0, NQT)])
        pltpu.sync_copy(zden_r.at[pl.ds(r0, NQT)], den_sh.at[pl.ds(r0, NQT)])
        plsc.subcore_barrier()
        _edge_sweep(h * NQ, srcv, dstv, esv, edv, z_r, acc_sh, den_sh, bufs)
        plsc.subcore_barrier()
        pltpu.sync_copy(acc_sh.at[pl.ds(r0, NQT)],
                        acc_r.at[h, pl.ds(r0, NQT)])
        pltpu.sync_copy(den_sh.at[pl.ds(r0, NQT)],
                        den_r.at[h, pl.ds(r0, NQT)])
        plsc.subcore_barrier()
        return cc
    lax.fori_loop(0, HEADS // NC, head, 0)


def _sc_scratch():
    buf = [
        pltpu.VMEM((CE,), f32),
        pltpu.VMEM((CE,), i32),
        pltpu.VMEM((CE, HID), f32),
    ]
    sems = [pltpu.SemaphoreType.DMA] * 9
    return ([
        pltpu.VMEM((NCH, CE), i32),
        pltpu.VMEM((NCH, CE), i32),
        pltpu.VMEM((NQ,), f32),
        pltpu.VMEM((NQ,), f32),
    ] + buf * 3 + [
        pltpu.VMEM_SHARED((NQ, HID), f32),
        pltpu.VMEM_SHARED((NQ,), f32),
    ] + sems)


@functools.cache
def _k2():
    return pl.kernel(
        _k2_body,
        out_type=[
            jax.ShapeDtypeStruct((HEADS, NQ, HID), f32),
            jax.ShapeDtypeStruct((HEADS, NQ), f32),
        ],
        mesh=plsc.VectorSubcoreMesh(core_axis_name="c", subcore_axis_name="s",
                                    num_cores=NC, num_subcores=NS),
        compiler_params=pltpu.CompilerParams(
            use_tc_tiling_on_sc=False, needs_layout_passes=False),
        scratch_types=_sc_scratch(),
    )


def _k3_body(acc_r, den_r, w2_r, wss_r, a2_r, z2_r, e2_r, hss_r):
    z2 = jnp.zeros((NB, OUT_DIM), f32)
    hss = jnp.zeros((NB, NUM_PAR), f32)
    for hd in range(HEADS):
        x = acc_r[hd] / (den_r[hd][:, None] + 1e-9)
        hm = jnp.where(x > 0, x, (jnp.exp(x) - 1.0))
        z2 += jnp.dot(hm, w2_r[hd * HID:(hd + 1) * HID, :],
                      preferred_element_type=f32)
        hss += jnp.dot(hm, wss_r[hd * HID:(hd + 1) * HID, :],
                       preferred_element_type=f32)
    z2_r[0] = z2[:, :HID]
    z2_r[1] = z2[:, HID:]
    hss_r[...] = hss
    e2_r[...] = lax.dot_general(a2_r[...], z2, (((0,), (1,)), ((), ())))


def _k3(acc1, den1, W2f, Wss, A2):
    return pl.pallas_call(
        _k3_body,
        grid=(GRID,),
        in_specs=[
            pl.BlockSpec((HEADS, NB, HID), lambda i: (0, i, 0)),
            pl.BlockSpec((HEADS, NB), lambda i: (0, i)),
            pl.BlockSpec((HEADS * HID, OUT_DIM), lambda i: (0, 0)),
            pl.BlockSpec((HEADS * HID, NUM_PAR), lambda i: (0, 0)),
            pl.BlockSpec((OUT_DIM, 8), lambda i: (0, 0)),
        ],
        out_specs=[
            pl.BlockSpec((NC, NB, HID), lambda i: (0, i, 0)),
            pl.BlockSpec((8, NB), lambda i: (0, i)),
            pl.BlockSpec((NB, NUM_PAR), lambda i: (i, 0)),
        ],
        out_shape=[
            jax.ShapeDtypeStruct((NC, NQ, HID), f32),
            jax.ShapeDtypeStruct((8, NQ), f32),
            jax.ShapeDtypeStruct((NQ, NUM_PAR), f32),
        ],
    )(acc1, den1, W2f, Wss, A2)


def _k4_body(src_r, dst_r, e2_r, z_r, zacc_r, zden_r, acc_r, den_r,
             srcv, dstv, esv, edv,
             exv0, idxv0, zb0, exv1, idxv1, zb1, exv2, idxv2, zb2,
             acc_sh, den_sh,
             sg0, sa0, sd0, sg1, sa1, sd1, sg2, sa2, sd2):
    c = lax.axis_index("c")
    s = lax.axis_index("s")
    pltpu.sync_copy(src_r.at[s], srcv)
    pltpu.sync_copy(dst_r.at[s], dstv)
    pltpu.sync_copy(e2_r.at[0], esv)
    pltpu.sync_copy(e2_r.at[1], edv)
    r0 = s * NQT
    pltpu.sync_copy(zacc_r.at[pl.ds(r0, NQT)], acc_sh.at[pl.ds(r0, NQT)])
    pltpu.sync_copy(zden_r.at[pl.ds(r0, NQT)], den_sh.at[pl.ds(r0, NQT)])
    plsc.subcore_barrier()
    bufs = [(exv0, idxv0, zb0, sg0, sa0, sd0),
            (exv1, idxv1, zb1, sg1, sa1, sd1),
            (exv2, idxv2, zb2, sg2, sa2, sd2)]
    _edge_sweep(c * NQ, srcv, dstv, esv, edv, z_r, acc_sh, den_sh, bufs)
    plsc.subcore_barrier()
    pltpu.sync_copy(acc_sh.at[pl.ds(r0, NQT)], acc_r.at[c, pl.ds(r0, NQT)])
    pltpu.sync_copy(den_sh.at[pl.ds(r0, NQT)], den_r.at[c, pl.ds(r0, NQT)])


@functools.cache
def _k4():
    return pl.kernel(
        _k4_body,
        out_type=[
            jax.ShapeDtypeStruct((NC, NQ, HID), f32),
            jax.ShapeDtypeStruct((NC, NQ), f32),
        ],
        mesh=plsc.VectorSubcoreMesh(core_axis_name="c", subcore_axis_name="s",
                                    num_cores=NC, num_subcores=NS),
        compiler_params=pltpu.CompilerParams(
            use_tc_tiling_on_sc=False, needs_layout_passes=False),
        scratch_types=_sc_scratch(),
    )


def _k5_body(acc_r, den_r, h2_r):
    x = jnp.concatenate([acc_r[0], acc_r[1]], axis=1)
    x = x / (den_r[0][:, None] + 1e-9)
    h2_r[...] = jnp.where(x > 0, x, (jnp.exp(x) - 1.0))


def _k5(acc2, den2):
    return pl.pallas_call(
        _k5_body,
        grid=(GRID,),
        in_specs=[
            pl.BlockSpec((NC, NB, HID), lambda i: (0, i, 0)),
            pl.BlockSpec((NC, NB), lambda i: (0, i)),
        ],
        out_specs=pl.BlockSpec((NB, OUT_DIM), lambda i: (i, 0)),
        out_shape=jax.ShapeDtypeStruct((NQ, OUT_DIM), f32),
    )(acc2, den2)


def kernel(h, edge_index, snorm_n, snorm_e, W1, a1_src, a1_dst,
           W2, a2_src, a2_dst, W_ss):
    src = edge_index[0]
    dst = edge_index[1]
    pad = EP - E
    srcp = jnp.concatenate([src, jnp.zeros((pad,), i32)])
    dstp = jnp.concatenate([dst, jnp.full((pad,), N, i32)])
    src2 = srcp.reshape(NS, NCH, CE)
    dst2 = dstp.reshape(NS, NCH, CE)

    hq = jnp.pad(h, ((0, NQ - N), (0, 0)))
    W1f = W1.reshape(IN_DIM, HEADS * HID)
    eye_rep = jnp.repeat(jnp.eye(HEADS, dtype=f32), HID, axis=0)
    A1 = jnp.concatenate([eye_rep * a1_src.reshape(-1, 1),
                          eye_rep * a1_dst.reshape(-1, 1)], axis=1)
    W2f = W2.reshape(HEADS * HID, OUT_DIM)
    A2 = jnp.concatenate(
        [a2_src.T, a2_dst.T, jnp.zeros((OUT_DIM, 6), f32)], axis=1)

    zacc = jnp.zeros((NQ, HID), f32)
    zden = jnp.zeros((NQ,), f32)

    z1, e1 = _k1(hq, W1f, A1)
    acc1, den1 = _k2()(src2, dst2, e1, z1.reshape(HEADS * NQ, HID),
                       zacc, zden)
    z2c, e2, hss = _k3(acc1, den1, W2f, W_ss, A2)
    acc2, den2 = _k4()(src2, dst2, e2, z2c.reshape(NC * NQ, HID),
                       zacc, zden)
    h2 = _k5(acc2, den2)
    return (h2[:N], hss[:N])


# speedup vs baseline: 1.4722x; 1.4722x over previous
import functools
import jax
import jax.numpy as jnp
from jax import lax
from jax.experimental import pallas as pl
from jax.experimental.pallas import tpu as pltpu
from jax.experimental.pallas import tpu_sc as plsc

N = 10000
E = 320000
IN_DIM = 128
HID = 64
OUT_DIM = 128
HEADS = 8
NUM_PAR = 32

NC = 2
NS = 16
CE = 128
NCH = 159
EP = NS * NCH * CE
NQ = 10240
NQT = NQ // NS
NB = 1024
GRID = NQ // NB

f32 = jnp.float32
i32 = jnp.int32


def _k1_body(h_ref, w1_ref, a1_ref, z_ref, e1_ref):
    z = jnp.dot(h_ref[...], w1_ref[...], preferred_element_type=f32)
    e1_ref[...] = lax.dot_general(a1_ref[...], z, (((0,), (1,)), ((), ())))
    for hd in range(HEADS):
        z_ref[hd] = z[:, hd * HID:(hd + 1) * HID]


def _k1(h, W1f, A1):
    return pl.pallas_call(
        _k1_body,
        grid=(GRID,),
        in_specs=[
            pl.BlockSpec((NB, IN_DIM), lambda i: (i, 0)),
            pl.BlockSpec((IN_DIM, HEADS * HID), lambda i: (0, 0)),
            pl.BlockSpec((HEADS * HID, 2 * HEADS), lambda i: (0, 0)),
        ],
        out_specs=[
            pl.BlockSpec((HEADS, NB, HID), lambda i: (0, i, 0)),
            pl.BlockSpec((2 * HEADS, NB), lambda i: (0, i)),
        ],
        out_shape=[
            jax.ShapeDtypeStruct((HEADS, NQ, HID), f32),
            jax.ShapeDtypeStruct((2 * HEADS, NQ), f32),
        ],
    )(h, W1f, A1)


def _edge_sweep(hN, srcv, dstv, esv, edv, z_r, acc_sh, den_sh, bufs):
    def front(g, b):
        exv, idxv, zb, sg, sa, sd = bufs[b]
        for j in range(CE // 16):
            sv = srcv[g, pl.ds(j * 16, 16)]
            dv = dstv[g, pl.ds(j * 16, 16)]
            e = plsc.load_gather(esv, [sv]) + plsc.load_gather(edv, [dv])
            e = jnp.where(e >= 0, e, 0.2 * e)
            exv[pl.ds(j * 16, 16)] = jnp.exp(e)
            idxv[pl.ds(j * 16, 16)] = sv + hN
        pltpu.async_copy(z_r.at[idxv], zb, sg)

    def wait_scat(b, g):
        exv, idxv, zb, sg, sa, sd = bufs[b]
        pltpu.make_async_copy(zb, acc_sh.at[dstv.at[g]], sa).wait()
        pltpu.make_async_copy(exv, den_sh.at[dstv.at[g]], sd).wait()

    def back(g, b):
        exv, idxv, zb, sg, sa, sd = bufs[b]
        pltpu.make_async_copy(z_r.at[idxv], zb, sg).wait()
        for jj in range(CE // 16):
            ex16 = exv[pl.ds(jj * 16, 16)]
            for rr in range(16):
                sc = ex16[rr]
                r = jj * 16 + rr
                for k in range(HID // 16):
                    zb[r, pl.ds(k * 16, 16)] = zb[r, pl.ds(k * 16, 16)] * sc
        pltpu.async_copy(zb, acc_sh.at[dstv.at[g]], sa, add=True)
        pltpu.async_copy(exv, den_sh.at[dstv.at[g]], sd, add=True)

    front(0, 0)
    front(1, 1)
    back(0, 0)
    front(2, 2)
    back(1, 1)

    def main(p, cc):
        for off in range(3):
            g = 3 * p + 3 + off
            wait_scat(off, g)
            front(g, off)
            back(g - 1, (off + 2) % 3)
        return cc
    lax.fori_loop(0, (NCH - 3) // 3, main, 0)
    back(NCH - 1, (NCH - 1) % 3)
    for b in range(3):
        wait_scat(b, 0)


def _k2_body(src_r, dst_r, e1_r, z_r, zacc_r, zden_r, acc_r, den_r,
             srcv, dstv, esv, edv,
             exv0, idxv0, zb0, exv1, idxv1, zb1, exv2, idxv2, zb2,
             acc_sh, den_sh,
             sg0, sa0, sd0, sg1, sa1, sd1, sg2, sa2, sd2):
    c = lax.axis_index("c")
    s = lax.axis_index("s")
    pltpu.sync_copy(src_r.at[s], srcv)
    pltpu.sync_copy(dst_r.at[s], dstv)
    r0 = s * NQT
    bufs = [(exv0, idxv0, zb0, sg0, sa0, sd0),
            (exv1, idxv1, zb1, sg1, sa1, sd1),
            (exv2, idxv2, zb2, sg2, sa2, sd2)]

    def head(hi, cc):
        h = c * (HEADS // NC) + hi
        pltpu.sync_copy(e1_r.at[h], esv)
        pltpu.sync_copy(e1_r.at[HEADS + h], edv)
        pltpu.sync_copy(zacc_r.at[pl.ds(r0, NQT)], acc_sh.at[pl.ds(r0, NQT)])
        pltpu.sync_copy(zden_r.at[pl.ds(r0, NQT)], den_sh.at[pl.ds(r0, NQT)])
        plsc.subcore_barrier()
        _edge_sweep(h * NQ, srcv, dstv, esv, edv, z_r, acc_sh, den_sh, bufs)
        plsc.subcore_barrier()
        pltpu.sync_copy(acc_sh.at[pl.ds(r0, NQT)],
                        acc_r.at[h, pl.ds(r0, NQT)])
        pltpu.sync_copy(den_sh.at[pl.ds(r0, NQT)],
                        den_r.at[h, pl.ds(r0, NQT)])
        plsc.subcore_barrier()
        return cc
    lax.fori_loop(0, HEADS // NC, head, 0)


def _sc_scratch():
    buf = [
        pltpu.VMEM((CE,), f32),
        pltpu.VMEM((CE,), i32),
        pltpu.VMEM((CE, HID), f32),
    ]
    sems = [pltpu.SemaphoreType.DMA] * 9
    return ([
        pltpu.VMEM((NCH, CE), i32),
        pltpu.VMEM((NCH, CE), i32),
        pltpu.VMEM((NQ,), f32),
        pltpu.VMEM((NQ,), f32),
    ] + buf * 3 + [
        pltpu.VMEM_SHARED((NQ, HID), f32),
        pltpu.VMEM_SHARED((NQ,), f32),
    ] + sems)


@functools.cache
def _k2():
    return pl.kernel(
        _k2_body,
        out_type=[
            jax.ShapeDtypeStruct((HEADS, NQ, HID), f32),
            jax.ShapeDtypeStruct((HEADS, NQ), f32),
        ],
        mesh=plsc.VectorSubcoreMesh(core_axis_name="c", subcore_axis_name="s",
                                    num_cores=NC, num_subcores=NS),
        compiler_params=pltpu.CompilerParams(
            use_tc_tiling_on_sc=False, needs_layout_passes=False),
        scratch_types=_sc_scratch(),
    )


def _k3_body(acc_r, den_r, w2_r, wss_r, a2_r, z2_r, e2_r, hss_r):
    z2 = jnp.zeros((NB, OUT_DIM), f32)
    hss = jnp.zeros((NB, NUM_PAR), f32)
    for hd in range(HEADS):
        x = acc_r[hd] / (den_r[hd][:, None] + 1e-9)
        hm = jnp.where(x > 0, x, (jnp.exp(x) - 1.0))
        z2 += jnp.dot(hm, w2_r[hd * HID:(hd + 1) * HID, :],
                      preferred_element_type=f32)
        hss += jnp.dot(hm, wss_r[hd * HID:(hd + 1) * HID, :],
                       preferred_element_type=f32)
    z2_r[0] = z2[:, :HID]
    z2_r[1] = z2[:, HID:]
    hss_r[...] = hss
    e2_r[...] = lax.dot_general(a2_r[...], z2, (((0,), (1,)), ((), ())))


def _k3(acc1, den1, W2f, Wss, A2):
    return pl.pallas_call(
        _k3_body,
        grid=(GRID,),
        in_specs=[
            pl.BlockSpec((HEADS, NB, HID), lambda i: (0, i, 0)),
            pl.BlockSpec((HEADS, NB), lambda i: (0, i)),
            pl.BlockSpec((HEADS * HID, OUT_DIM), lambda i: (0, 0)),
            pl.BlockSpec((HEADS * HID, NUM_PAR), lambda i: (0, 0)),
            pl.BlockSpec((OUT_DIM, 8), lambda i: (0, 0)),
        ],
        out_specs=[
            pl.BlockSpec((NC, NB, HID), lambda i: (0, i, 0)),
            pl.BlockSpec((8, NB), lambda i: (0, i)),
            pl.BlockSpec((NB, NUM_PAR), lambda i: (i, 0)),
        ],
        out_shape=[
            jax.ShapeDtypeStruct((NC, NQ, HID), f32),
            jax.ShapeDtypeStruct((8, NQ), f32),
            jax.ShapeDtypeStruct((NQ, NUM_PAR), f32),
        ],
    )(acc1, den1, W2f, Wss, A2)


def _k4_body(src_r, dst_r, e2_r, z_r, zacc_r, zden_r, acc_r, den_r,
             srcv, dstv, esv, edv,
             exv0, idxv0, zb0, exv1, idxv1, zb1, exv2, idxv2, zb2,
             acc_sh, den_sh,
             sg0, sa0, sd0, sg1, sa1, sd1, sg2, sa2, sd2):
    c = lax.axis_index("c")
    s = lax.axis_index("s")
    pltpu.sync_copy(src_r.at[s], srcv)
    pltpu.sync_copy(dst_r.at[s], dstv)
    pltpu.sync_copy(e2_r.at[0], esv)
    pltpu.sync_copy(e2_r.at[1], edv)
    r0 = s * NQT
    pltpu.sync_copy(zacc_r.at[pl.ds(r0, NQT)], acc_sh.at[pl.ds(r0, NQT)])
    pltpu.sync_copy(zden_r.at[pl.ds(r0, NQT)], den_sh.at[pl.ds(r0, NQT)])
    plsc.subcore_barrier()
    bufs = [(exv0, idxv0, zb0, sg0, sa0, sd0),
            (exv1, idxv1, zb1, sg1, sa1, sd1),
            (exv2, idxv2, zb2, sg2, sa2, sd2)]
    _edge_sweep(c * NQ, srcv, dstv, esv, edv, z_r, acc_sh, den_sh, bufs)
    plsc.subcore_barrier()
    pltpu.sync_copy(acc_sh.at[pl.ds(r0, NQT)], acc_r.at[c, pl.ds(r0, NQT)])
    pltpu.sync_copy(den_sh.at[pl.ds(r0, NQT)], den_r.at[c, pl.ds(r0, NQT)])


@functools.cache
def _k4():
    return pl.kernel(
        _k4_body,
        out_type=[
            jax.ShapeDtypeStruct((NC, NQ, HID), f32),
            jax.ShapeDtypeStruct((NC, NQ), f32),
        ],
        mesh=plsc.VectorSubcoreMesh(core_axis_name="c", subcore_axis_name="s",
                                    num_cores=NC, num_subcores=NS),
        compiler_params=pltpu.CompilerParams(
            use_tc_tiling_on_sc=False, needs_layout_passes=False),
        scratch_types=_sc_scratch(),
    )


def _k5_body(acc_r, den_r, h2_r):
    x = jnp.concatenate([acc_r[0], acc_r[1]], axis=1)
    x = x / (den_r[0][:, None] + 1e-9)
    h2_r[...] = jnp.where(x > 0, x, (jnp.exp(x) - 1.0))


def _k5(acc2, den2):
    return pl.pallas_call(
        _k5_body,
        grid=(GRID,),
        in_specs=[
            pl.BlockSpec((NC, NB, HID), lambda i: (0, i, 0)),
            pl.BlockSpec((NC, NB), lambda i: (0, i)),
        ],
        out_specs=pl.BlockSpec((NB, OUT_DIM), lambda i: (i, 0)),
        out_shape=jax.ShapeDtypeStruct((NQ, OUT_DIM), f32),
    )(acc2, den2)


def kernel(h, edge_index, snorm_n, snorm_e, W1, a1_src, a1_dst,
           W2, a2_src, a2_dst, W_ss):
    src = edge_index[0]
    dst = edge_index[1]
    pad = EP - E
    srcp = jnp.concatenate([src, jnp.zeros((pad,), i32)])
    dstp = jnp.concatenate([dst, jnp.full((pad,), N, i32)])
    src2 = srcp.reshape(NS, NCH, CE)
    dst2 = dstp.reshape(NS, NCH, CE)

    hq = jnp.pad(h, ((0, NQ - N), (0, 0)))
    W1f = W1.reshape(IN_DIM, HEADS * HID)
    eye_rep = jnp.repeat(jnp.eye(HEADS, dtype=f32), HID, axis=0)
    A1 = jnp.concatenate([eye_rep * a1_src.reshape(-1, 1),
                          eye_rep * a1_dst.reshape(-1, 1)], axis=1)
    W2f = W2.reshape(HEADS * HID, OUT_DIM)
    A2 = jnp.concatenate(
        [a2_src.T, a2_dst.T, jnp.zeros((OUT_DIM, 6), f32)], axis=1)

    zacc = jnp.zeros((NQ, HID), f32)
    zden = jnp.zeros((NQ,), f32)

    z1, e1 = _k1(hq, W1f, A1)
    acc1, den1 = _k2()(src2, dst2, e1, z1.reshape(HEADS * NQ, HID),
                       zacc, zden)
    z2c, e2, hss = _k3(acc1, den1, W2f, W_ss, A2)
    acc2, den2 = _k4()(src2, dst2, e2, z2c.reshape(NC * NQ, HID),
                       zacc, zden)
    h2 = _k5(acc2, den2)
    return (h2[:N], hss[:N])


# speedup vs baseline: 2.1931x; 1.4897x over previous
import functools
import jax
import jax.numpy as jnp
from jax import lax
from jax.experimental import pallas as pl
from jax.experimental.pallas import tpu as pltpu
from jax.experimental.pallas import tpu_sc as plsc

N = 10000
E = 320000
IN_DIM = 128
HID = 64
OUT_DIM = 128
HEADS = 8
NUM_PAR = 32

NC = 2
NS = 16
CE = 128
NCH = 158
EP = NS * NCH * CE
NQ = 10240
NQT = NQ // NS
NB = 1024
GRID = NQ // NB

f32 = jnp.float32
bf16 = jnp.bfloat16
i32 = jnp.int32


def _k1_body(h_ref, w1_ref, a1_ref, z_ref, e1_ref):
    z = jnp.dot(h_ref[...], w1_ref[...], preferred_element_type=f32)
    e1_ref[...] = lax.dot_general(a1_ref[...], z, (((0,), (1,)), ((), ())))
    for hd in range(HEADS):
        z_ref[hd] = z[:, hd * HID:(hd + 1) * HID].astype(bf16)


def _k1(h, W1f, A1):
    return pl.pallas_call(
        _k1_body,
        grid=(GRID,),
        in_specs=[
            pl.BlockSpec((NB, IN_DIM), lambda i: (i, 0)),
            pl.BlockSpec((IN_DIM, HEADS * HID), lambda i: (0, 0)),
            pl.BlockSpec((HEADS * HID, 2 * HEADS), lambda i: (0, 0)),
        ],
        out_specs=[
            pl.BlockSpec((HEADS, NB, HID), lambda i: (0, i, 0)),
            pl.BlockSpec((2 * HEADS, NB), lambda i: (0, i)),
        ],
        out_shape=[
            jax.ShapeDtypeStruct((HEADS, NQ, HID), bf16),
            jax.ShapeDtypeStruct((2 * HEADS, NQ), f32),
        ],
    )(h, W1f, A1)


def _edge_sweep(hN, srcv, dstv, esv, edv, z_r, acc_sh, den_sh, bufs):
    def front(g, b, first=False):
        exv, idxv, zb, zs, sg, sa, sd = bufs[b]
        exs, idxs = [], []
        for j in range(CE // 16):
            sv = srcv[g, pl.ds(j * 16, 16)]
            dv = dstv[g, pl.ds(j * 16, 16)]
            e = plsc.load_gather(esv, [sv]) + plsc.load_gather(edv, [dv])
            e = jnp.where(e >= 0, e, 0.2 * e)
            exs.append(jnp.exp(e))
            idxs.append(sv + hN)
        if not first:
            pltpu.make_async_copy(zs, acc_sh.at[dstv.at[g]], sa).wait()
            pltpu.make_async_copy(exv, den_sh.at[dstv.at[g]], sd).wait()
        for j in range(CE // 16):
            exv[pl.ds(j * 16, 16)] = exs[j]
            idxv[pl.ds(j * 16, 16)] = idxs[j]
        pltpu.async_copy(z_r.at[idxv], zb, sg)

    def wait_scat(b, g):
        exv, idxv, zb, zs, sg, sa, sd = bufs[b]
        pltpu.make_async_copy(zs, acc_sh.at[dstv.at[g]], sa).wait()
        pltpu.make_async_copy(exv, den_sh.at[dstv.at[g]], sd).wait()

    def back(g, b):
        exv, idxv, zb, zs, sg, sa, sd = bufs[b]
        pltpu.make_async_copy(z_r.at[idxv], zb, sg).wait()
        for jj in range(CE // 16):
            ex16 = exv[pl.ds(jj * 16, 16)]
            for rr in range(16):
                sc = ex16[rr]
                r = jj * 16 + rr
                for q in range(HID // 32):
                    u = zb[r, pl.ds(q * 32, 32)]
                    lo, hi = plsc.unpack(
                        u, format=plsc.PackFormat.INTERLEAVED)
                    zs[r, pl.ds(q * 32, 16)] = lo * sc
                    zs[r, pl.ds(q * 32 + 16, 16)] = hi * sc
        pltpu.async_copy(zs, acc_sh.at[dstv.at[g]], sa, add=True)
        pltpu.async_copy(exv, den_sh.at[dstv.at[g]], sd, add=True)

    front(0, 0, first=True)
    front(1, 1, first=True)
    back(0, 0)
    front(2, 0)
    back(1, 1)

    def main(p, cc):
        for off in range(2):
            g = 2 * p + 3 + off
            front(g, 1 - off)
            back(g - 1, off)
        return cc
    lax.fori_loop(0, (NCH - 4) // 2, main, 0)
    front(NCH - 1, 1)
    back(NCH - 2, 0)
    back(NCH - 1, 1)
    for b in range(2):
        wait_scat(b, 0)


def _k2_body(src_r, dst_r, e1_r, z_r, zacc_r, zden_r, acc_r, den_r,
             srcv, dstv, esv, edv,
             exv0, idxv0, zb0, zs0, exv1, idxv1, zb1, zs1,
             acc_sh, den_sh,
             sg0, sa0, sd0, sg1, sa1, sd1):
    c = lax.axis_index("c")
    s = lax.axis_index("s")
    pltpu.sync_copy(src_r.at[s], srcv)
    pltpu.sync_copy(dst_r.at[s], dstv)
    r0 = s * NQT
    bufs = [(exv0, idxv0, zb0, zs0, sg0, sa0, sd0),
            (exv1, idxv1, zb1, zs1, sg1, sa1, sd1)]

    def head(hi, cc):
        h = c * (HEADS // NC) + hi
        pltpu.sync_copy(e1_r.at[h], esv)
        pltpu.sync_copy(e1_r.at[HEADS + h], edv)
        pltpu.sync_copy(zacc_r.at[pl.ds(r0, NQT)], acc_sh.at[pl.ds(r0, NQT)])
        pltpu.sync_copy(zden_r.at[pl.ds(r0, NQT)], den_sh.at[pl.ds(r0, NQT)])
        plsc.subcore_barrier()
        _edge_sweep(h * NQ, srcv, dstv, esv, edv, z_r, acc_sh, den_sh, bufs)
        plsc.subcore_barrier()
        pltpu.sync_copy(acc_sh.at[pl.ds(r0, NQT)],
                        acc_r.at[h, pl.ds(r0, NQT)])
        pltpu.sync_copy(den_sh.at[pl.ds(r0, NQT)],
                        den_r.at[h, pl.ds(r0, NQT)])
        plsc.subcore_barrier()
        return cc
    lax.fori_loop(0, HEADS // NC, head, 0)


def _sc_scratch():
    buf = [
        pltpu.VMEM((CE,), f32),
        pltpu.VMEM((CE,), i32),
        pltpu.VMEM((CE, HID), bf16),
        pltpu.VMEM((CE, HID), f32),
    ]
    sems = [pltpu.SemaphoreType.DMA] * 6
    return ([
        pltpu.VMEM((NCH, CE), i32),
        pltpu.VMEM((NCH, CE), i32),
        pltpu.VMEM((NQ,), f32),
        pltpu.VMEM((NQ,), f32),
    ] + buf * 2 + [
        pltpu.VMEM_SHARED((NQ, HID), f32),
        pltpu.VMEM_SHARED((NQ,), f32),
    ] + sems)


@functools.cache
def _k2():
    return pl.kernel(
        _k2_body,
        out_type=[
            jax.ShapeDtypeStruct((HEADS, NQ, HID), f32),
            jax.ShapeDtypeStruct((HEADS, NQ), f32),
        ],
        mesh=plsc.VectorSubcoreMesh(core_axis_name="c", subcore_axis_name="s",
                                    num_cores=NC, num_subcores=NS),
        compiler_params=pltpu.CompilerParams(
            use_tc_tiling_on_sc=False, needs_layout_passes=False),
        scratch_types=_sc_scratch(),
    )


def _k3_body(acc_r, den_r, w2_r, wss_r, a2_r, z2_r, e2_r, hss_r):
    z2 = jnp.zeros((NB, OUT_DIM), f32)
    hss = jnp.zeros((NB, NUM_PAR), f32)
    for hd in range(HEADS):
        x = acc_r[hd] / (den_r[hd][:, None] + 1e-9)
        hm = jnp.where(x > 0, x, (jnp.exp(x) - 1.0))
        z2 += jnp.dot(hm, w2_r[hd * HID:(hd + 1) * HID, :],
                      preferred_element_type=f32)
        hss += jnp.dot(hm, wss_r[hd * HID:(hd + 1) * HID, :],
                       preferred_element_type=f32)
    z2_r[0] = z2[:, :HID].astype(bf16)
    z2_r[1] = z2[:, HID:].astype(bf16)
    hss_r[...] = hss
    e2_r[...] = lax.dot_general(a2_r[...], z2, (((0,), (1,)), ((), ())))


def _k3(acc1, den1, W2f, Wss, A2):
    return pl.pallas_call(
        _k3_body,
        grid=(GRID,),
        in_specs=[
            pl.BlockSpec((HEADS, NB, HID), lambda i: (0, i, 0)),
            pl.BlockSpec((HEADS, NB), lambda i: (0, i)),
            pl.BlockSpec((HEADS * HID, OUT_DIM), lambda i: (0, 0)),
            pl.BlockSpec((HEADS * HID, NUM_PAR), lambda i: (0, 0)),
            pl.BlockSpec((OUT_DIM, 8), lambda i: (0, 0)),
        ],
        out_specs=[
            pl.BlockSpec((NC, NB, HID), lambda i: (0, i, 0)),
            pl.BlockSpec((8, NB), lambda i: (0, i)),
            pl.BlockSpec((NB, NUM_PAR), lambda i: (i, 0)),
        ],
        out_shape=[
            jax.ShapeDtypeStruct((NC, NQ, HID), bf16),
            jax.ShapeDtypeStruct((8, NQ), f32),
            jax.ShapeDtypeStruct((N, NUM_PAR), f32),
        ],
    )(acc1, den1, W2f, Wss, A2)


def _k4_body(src_r, dst_r, e2_r, z_r, zacc_r, zden_r, h2_r,
             srcv, dstv, esv, edv,
             exv0, idxv0, zb0, zs0, exv1, idxv1, zb1, zs1,
             acc_sh, den_sh,
             sg0, sa0, sd0, sg1, sa1, sd1):
    c = lax.axis_index("c")
    s = lax.axis_index("s")
    pltpu.sync_copy(src_r.at[s], srcv)
    pltpu.sync_copy(dst_r.at[s], dstv)
    pltpu.sync_copy(e2_r.at[0], esv)
    pltpu.sync_copy(e2_r.at[1], edv)
    r0 = s * NQT
    pltpu.sync_copy(zacc_r.at[pl.ds(r0, NQT)], acc_sh.at[pl.ds(r0, NQT)])
    pltpu.sync_copy(zden_r.at[pl.ds(r0, NQT)], den_sh.at[pl.ds(r0, NQT)])
    plsc.subcore_barrier()
    bufs = [(exv0, idxv0, zb0, zs0, sg0, sa0, sd0),
            (exv1, idxv1, zb1, zs1, sg1, sa1, sd1)]
    _edge_sweep(c * NQ, srcv, dstv, esv, edv, z_r, acc_sh, den_sh, bufs)
    plsc.subcore_barrier()

    def finalize(t, cc):
        rows = r0 + t * CE
        pltpu.sync_copy(acc_sh.at[pl.ds(rows, CE)], zs0)
        pltpu.sync_copy(den_sh.at[pl.ds(rows, CE)], exv0)

        def norm(jj, cc2):
            d16 = exv0[pl.ds(jj * 16, 16)]
            w16 = 1.0 / (d16 + 1e-9)
            for rr in range(16):
                sc = w16[rr]
                r = jj * 16 + rr
                for k in range(HID // 16):
                    x = zs0[r, pl.ds(k * 16, 16)] * sc
                    zs0[r, pl.ds(k * 16, 16)] = jnp.where(
                        x > 0, x, jnp.exp(x) - 1.0)
            return cc2
        lax.fori_loop(0, CE // 16, norm, 0)

        @pl.when(rows + CE <= N)
        def _():
            pltpu.sync_copy(
                zs0, h2_r.at[pl.ds(rows, CE), pl.ds(c * HID, HID)])

        @pl.when(jnp.logical_and(rows < N, rows + CE > N))
        def _():
            pltpu.sync_copy(
                zs0.at[pl.ds(0, N % CE)],
                h2_r.at[pl.ds(rows, N % CE), pl.ds(c * HID, HID)])
        return cc
    lax.fori_loop(0, NQT // CE, finalize, 0)


@functools.cache
def _k4():
    return pl.kernel(
        _k4_body,
        out_type=jax.ShapeDtypeStruct((N, OUT_DIM), f32),
        mesh=plsc.VectorSubcoreMesh(core_axis_name="c", subcore_axis_name="s",
                                    num_cores=NC, num_subcores=NS),
        compiler_params=pltpu.CompilerParams(
            use_tc_tiling_on_sc=False, needs_layout_passes=False),
        scratch_types=_sc_scratch(),
    )


def kernel(h, edge_index, snorm_n, snorm_e, W1, a1_src, a1_dst,
           W2, a2_src, a2_dst, W_ss):
    pad = EP - E
    padblk = jnp.concatenate([jnp.zeros((1, pad), i32),
                              jnp.full((1, pad), N, i32)])
    ep = jnp.concatenate([edge_index, padblk], axis=1)
    src2 = ep[0].reshape(NS, NCH, CE)
    dst2 = ep[1].reshape(NS, NCH, CE)

    hq = jnp.pad(h, ((0, NQ - N), (0, 0)))
    p64 = 32 * (jnp.arange(64) // 32) + 16 * (jnp.arange(64) % 2) \
        + (jnp.arange(64) % 32) // 2
    cp1 = (jnp.arange(HEADS * HID) // HID) * HID + p64[jnp.arange(
        HEADS * HID) % HID]
    cp2 = (jnp.arange(OUT_DIM) // HID) * HID + p64[jnp.arange(
        OUT_DIM) % HID]
    W1f = W1.reshape(IN_DIM, HEADS * HID)[:, cp1]
    eye_rep = jnp.repeat(jnp.eye(HEADS, dtype=f32), HID, axis=0)
    A1 = jnp.concatenate([eye_rep * a1_src.reshape(-1, 1),
                          eye_rep * a1_dst.reshape(-1, 1)], axis=1)[cp1, :]
    W2f = W2.reshape(HEADS * HID, OUT_DIM)[:, cp2]
    A2 = jnp.concatenate(
        [a2_src.T, a2_dst.T, jnp.zeros((OUT_DIM, 6), f32)], axis=1)[cp2, :]

    zacc = jnp.zeros((NQ, HID), f32)
    zden = jnp.zeros((NQ,), f32)

    z1, e1 = _k1(hq, W1f, A1)
    acc1, den1 = _k2()(src2, dst2, e1, z1.reshape(HEADS * NQ, HID),
                       zacc, zden)
    z2c, e2, hss = _k3(acc1, den1, W2f, W_ss, A2)
    h2 = _k4()(src2, dst2, e2, z2c.reshape(NC * NQ, HID), zacc, zden)
    return (h2, hss)
